# baseline (speedup 1.0000x reference)
import functools

import jax
import jax.numpy as jnp
from jax import lax
from jax.experimental import pallas as pl
from jax.experimental.pallas import tpu as pltpu
from jax.experimental.pallas import tpu_sc as plsc

B = 16384
D = 128
NC = 2
NS = 16
L = 16
NW = NC * NS
BPW = B // NW
CH = 128
NCH = BPW // CH


def _mf_body(user_table, item_table, users_r, items_r, out_hbm,
             uidx, iidx, urows, irows, out_v,
             sem_u0, sem_i0, sem_u1, sem_i1):
    wid = lax.axis_index("s") * NC + lax.axis_index("c")

    pltpu.sync_copy(users_r.at[wid], uidx)
    pltpu.sync_copy(items_r.at[wid], iidx)

    sems_u = (sem_u0, sem_u1)
    sems_i = (sem_i0, sem_i1)

    def start(c):
        b = c % 2
        cu = pltpu.make_async_copy(user_table.at[uidx.at[c]], urows.at[b],
                                   sems_u[b])
        ci = pltpu.make_async_copy(item_table.at[iidx.at[c]], irows.at[b],
                                   sems_i[b])
        cu.start()
        ci.start()
        return cu, ci

    row_iota = lax.iota(jnp.int32, L)
    pending = start(0)
    for c in range(NCH):
        nxt = start(c + 1) if c + 1 < NCH else None
        pending[0].wait()
        pending[1].wait()
        b = c % 2
        ub = urows.at[b]
        ib = irows.at[b]

        def gbody(g, _, ub=ub, ib=ib, c=c):
            rows = g * L + row_iota
            accs = [jnp.zeros((L,), jnp.float32) for _ in range(4)]

            def dbody(t, accs, ub=ub, ib=ib, rows=rows):
                a0, a1, a2, a3 = accs
                base = t * 4
                cols = [jnp.full((L,), base + q, jnp.int32) for q in range(4)]
                a0 = a0 + plsc.load_gather(ub, [rows, cols[0]]) * plsc.load_gather(ib, [rows, cols[0]])
                a1 = a1 + plsc.load_gather(ub, [rows, cols[1]]) * plsc.load_gather(ib, [rows, cols[1]])
                a2 = a2 + plsc.load_gather(ub, [rows, cols[2]]) * plsc.load_gather(ib, [rows, cols[2]])
                a3 = a3 + plsc.load_gather(ub, [rows, cols[3]]) * plsc.load_gather(ib, [rows, cols[3]])
                return (a0, a1, a2, a3)

            accs = lax.fori_loop(0, D // 4, dbody, tuple(accs), unroll=4)
            out16 = (accs[0] + accs[1]) + (accs[2] + accs[3])
            out_v[pl.ds(c * CH + g * L, L)] = out16
            return 0

        lax.fori_loop(0, CH // L, gbody, 0)
        pending = nxt

    pltpu.sync_copy(out_v, out_hbm.at[wid])


@jax.jit
def _run(users, items, user_table, item_table):
    users_r = users.astype(jnp.int32).reshape(NW, NCH, CH)
    items_r = items.astype(jnp.int32).reshape(NW, NCH, CH)
    mesh = plsc.VectorSubcoreMesh(core_axis_name="c", subcore_axis_name="s")
    k = pl.kernel(
        _mf_body,
        out_type=jax.ShapeDtypeStruct((NW, BPW), jnp.float32),
        mesh=mesh,
        compiler_params=pltpu.CompilerParams(needs_layout_passes=False),
        scratch_types=[
            pltpu.VMEM((NCH, CH), jnp.int32),
            pltpu.VMEM((NCH, CH), jnp.int32),
            pltpu.VMEM((2, CH, D), jnp.float32),
            pltpu.VMEM((2, CH, D), jnp.float32),
            pltpu.VMEM((BPW,), jnp.float32),
            pltpu.SemaphoreType.DMA,
            pltpu.SemaphoreType.DMA,
            pltpu.SemaphoreType.DMA,
            pltpu.SemaphoreType.DMA,
        ],
    )
    out = k(user_table, item_table, users_r, items_r)
    return out.reshape(B)


def kernel(users, items, user_table, item_table):
    return _run(users, items, user_table, item_table)


# speedup vs baseline: 2.2601x; 2.2601x over previous
import functools

import jax
import jax.numpy as jnp
from jax import lax
from jax.experimental import pallas as pl
from jax.experimental.pallas import tpu as pltpu
from jax.experimental.pallas import tpu_sc as plsc

B = 16384
D = 128
NC = 2
NS = 16
L = 16
NW = NC * NS
BPW = B // NW
CH = 128
NCH = BPW // CH


def _mf_body(user_table, item_table, users_r, items_r, out_hbm,
             uidx, iidx, urows, irows, psum, out_v,
             sem_u0, sem_i0, sem_u1, sem_i1):
    wid = lax.axis_index("s") * NC + lax.axis_index("c")

    pltpu.sync_copy(users_r.at[wid], uidx)
    pltpu.sync_copy(items_r.at[wid], iidx)

    sems_u = (sem_u0, sem_u1)
    sems_i = (sem_i0, sem_i1)

    def start(c):
        b = c % 2
        cu = pltpu.make_async_copy(user_table.at[uidx.at[c]], urows.at[b],
                                   sems_u[b])
        ci = pltpu.make_async_copy(item_table.at[iidx.at[c]], irows.at[b],
                                   sems_i[b])
        cu.start()
        ci.start()
        return cu, ci

    row_iota = lax.iota(jnp.int32, L)
    pending = start(0)
    for c in range(NCH):
        nxt = start(c + 1) if c + 1 < NCH else None
        pending[0].wait()
        pending[1].wait()
        b = c % 2
        ub = urows.at[b]
        ib = irows.at[b]

        def gbody(g, _, ub=ub, ib=ib, c=c):
            for j in range(L):
                r = g * L + j
                acc = ub[r, pl.ds(0, L)] * ib[r, pl.ds(0, L)]
                for k in range(1, D // L):
                    sl = pl.ds(k * L, L)
                    acc = acc + ub[r, sl] * ib[r, sl]
                psum[pl.ds(j * (L + 1), L)] = acc
            rows17 = row_iota * (L + 1)
            out16 = plsc.load_gather(psum, [rows17])
            for m in range(1, L):
                out16 = out16 + plsc.load_gather(psum, [rows17 + m])
            out_v[pl.ds(c * CH + g * L, L)] = out16
            return 0

        lax.fori_loop(0, CH // L, gbody, 0)
        pending = nxt

    pltpu.sync_copy(out_v, out_hbm.at[wid])


@jax.jit
def _run(users, items, user_table, item_table):
    users_r = users.astype(jnp.int32).reshape(NW, NCH, CH)
    items_r = items.astype(jnp.int32).reshape(NW, NCH, CH)
    mesh = plsc.VectorSubcoreMesh(core_axis_name="c", subcore_axis_name="s")
    k = pl.kernel(
        _mf_body,
        out_type=jax.ShapeDtypeStruct((NW, BPW), jnp.float32),
        mesh=mesh,
        compiler_params=pltpu.CompilerParams(needs_layout_passes=False),
        scratch_types=[
            pltpu.VMEM((NCH, CH), jnp.int32),
            pltpu.VMEM((NCH, CH), jnp.int32),
            pltpu.VMEM((2, CH, D), jnp.float32),
            pltpu.VMEM((2, CH, D), jnp.float32),
            pltpu.VMEM((L * (L + 1),), jnp.float32),
            pltpu.VMEM((BPW,), jnp.float32),
            pltpu.SemaphoreType.DMA,
            pltpu.SemaphoreType.DMA,
            pltpu.SemaphoreType.DMA,
            pltpu.SemaphoreType.DMA,
        ],
    )
    out = k(user_table, item_table, users_r, items_r)
    return out.reshape(B)


def kernel(users, items, user_table, item_table):
    return _run(users, items, user_table, item_table)
